# baseline (speedup 1.0000x reference)
import functools

import jax
import jax.numpy as jnp
from jax import lax
from jax.experimental import pallas as pl
from jax.experimental.pallas import tpu as pltpu
from jax.experimental.pallas import tpu_sc as plsc

D_MODEL = 1024
NUM_EMB = 4

_NC = 2
_NS = 16
_NW = _NC * _NS

_TOTAL = 4 * 8192
_BPW = _TOTAL // _NW

_GROWS = 576
_RG = 32
_NCHG = _GROWS // _RG

_FROWS = _BPW - _GROWS
_RF = 16
_NCHF = _FROWS // _RF


def _sc_body(ids_hbm, table_hbm, out_hbm, idx_v, table_v, gbuf, fbuf,
             gsemA, gsemB, ssemA, ssemB, fsemA, fsemB):
    sid = lax.axis_index("s")
    wid = sid * _NC + lax.axis_index("c")
    base = wid * _BPW

    pltpu.sync_copy(table_hbm.at[pl.ds(0, NUM_EMB)], table_v)
    pltpu.sync_copy(ids_hbm.at[pl.ds(base, _BPW)], idx_v)
    off = wid * NUM_EMB
    for j in range(_GROWS // 16):
        sl = pl.ds(j * 16, 16)
        idx_v[sl] = idx_v[sl] + off

    iota = lax.iota(jnp.int32, 16)
    gsems = (gsemA, gsemB)
    ssems = (ssemA, ssemB)
    fsems = (fsemA, fsemB)

    def g_gather(g, par):
        return pltpu.make_async_copy(
            table_hbm.at[idx_v.at[pl.ds(g * _RG, _RG)]],
            gbuf.at[par], gsems[par])

    def g_scat(g, par):
        return pltpu.make_async_copy(
            gbuf.at[par], out_hbm.at[pl.ds(base + g * _RG, _RG)], ssems[par])

    def f_scat(i, par):
        return pltpu.make_async_copy(
            fbuf.at[pl.ds(par * _RF, _RF)],
            out_hbm.at[pl.ds(base + _GROWS + i * _RF, _RF)], fsems[par])

    def step(i, carry):
        par_f = i % 2

        @pl.when(i >= 2)
        def _():
            @pl.when(par_f == 0)
            def _():
                f_scat(i - 2, 0).wait()

            @pl.when(par_f == 1)
            def _():
                f_scat(i - 2, 1).wait()

        @pl.when(i < _NCHG)
        def _():
            g = i
            par_g = g % 2

            @pl.when(g == 0)
            def _():
                g_gather(0, 0).start()

            @pl.when(par_g == 0)
            def _():
                g_gather(g, 0).wait()
                g_scat(g, 0).start()

            @pl.when(par_g == 1)
            def _():
                g_gather(g, 1).wait()
                g_scat(g, 1).start()

            @pl.when(g >= 1)
            def _():
                @pl.when(par_g == 0)
                def _():
                    g_scat(g - 1, 1).wait()

                @pl.when(par_g == 1)
                def _():
                    g_scat(g - 1, 0).wait()

            @pl.when(g + 1 < _NCHG)
            def _():
                @pl.when(par_g == 0)
                def _():
                    g_gather(g + 1, 1).start()

                @pl.when(par_g == 1)
                def _():
                    g_gather(g + 1, 0).start()

        @plsc.parallel_loop(0, _RF)
        def _(r):
            rid = plsc.load_gather(
                idx_v, [jnp.full((16,), 0, jnp.int32) + (_GROWS + i * _RF + r)])
            row2d = par_f * _RF + r
            for j in range(D_MODEL // 16):
                vals = plsc.load_gather(table_v, [rid, iota + (j * 16)])
                fbuf[row2d, pl.ds(j * 16, 16)] = vals

        @pl.when(par_f == 0)
        def _():
            f_scat(i, 0).start()

        @pl.when(par_f == 1)
        def _():
            f_scat(i, 1).start()

        return carry

    lax.fori_loop(0, _NCHF, step, 0)
    f_scat(_NCHF - 2, 0).wait()
    f_scat(_NCHF - 1, 1).wait()
    g_scat(_NCHG - 1, (_NCHG - 1) % 2).wait()


@jax.jit
def _sc_lookup(ids_flat, table_rep):
    mesh = plsc.VectorSubcoreMesh(
        core_axis_name="c", subcore_axis_name="s",
        num_cores=_NC, num_subcores=_NS)
    f = functools.partial(
        pl.kernel,
        out_type=jax.ShapeDtypeStruct((_TOTAL, D_MODEL), jnp.float32),
        mesh=mesh,
        scratch_types=[
            pltpu.VMEM((_BPW,), jnp.int32),
            pltpu.VMEM((NUM_EMB, D_MODEL), jnp.float32),
            pltpu.VMEM((2, _RG, D_MODEL), jnp.float32),
            pltpu.VMEM((2 * _RF, D_MODEL), jnp.float32),
            pltpu.SemaphoreType.DMA,
            pltpu.SemaphoreType.DMA,
            pltpu.SemaphoreType.DMA,
            pltpu.SemaphoreType.DMA,
            pltpu.SemaphoreType.DMA,
            pltpu.SemaphoreType.DMA,
        ],
        compiler_params=pltpu.CompilerParams(needs_layout_passes=False),
    )(_sc_body)
    return f(ids_flat, table_rep)


def kernel(postion_ids, table):
    B, S = postion_ids.shape
    ids_flat = postion_ids.reshape(B * S).astype(jnp.int32)
    table_rep = jnp.tile(table, (_NW, 1))
    out = _sc_lookup(ids_flat, table_rep)
    return out.reshape(B, S, D_MODEL)


# speedup vs baseline: 1.0261x; 1.0261x over previous
import functools

import jax
import jax.numpy as jnp
from jax import lax
from jax.experimental import pallas as pl
from jax.experimental.pallas import tpu as pltpu
from jax.experimental.pallas import tpu_sc as plsc

D_MODEL = 1024
NUM_EMB = 4

_NC = 2
_NS = 16
_NW = _NC * _NS

_TOTAL = 4 * 8192
_BPW = _TOTAL // _NW

_GROWS = 448
_RG = 32
_NCHG = _GROWS // _RG

_FROWS = _BPW - _GROWS
_RF = 16
_NCHF = _FROWS // _RF


def _sc_body(ids_hbm, table_hbm, out_hbm, idx_v, table_v, gbuf, fbuf,
             gsemA, gsemB, ssemA, ssemB, fsemA, fsemB):
    sid = lax.axis_index("s")
    wid = sid * _NC + lax.axis_index("c")
    base = wid * _BPW

    pltpu.sync_copy(table_hbm.at[pl.ds(0, NUM_EMB)], table_v)
    pltpu.sync_copy(ids_hbm.at[pl.ds(base, _BPW)], idx_v)
    off = wid * NUM_EMB
    for j in range(_GROWS // 16):
        sl = pl.ds(j * 16, 16)
        idx_v[sl] = idx_v[sl] + off

    iota = lax.iota(jnp.int32, 16)
    gsems = (gsemA, gsemB)
    ssems = (ssemA, ssemB)
    fsems = (fsemA, fsemB)

    def g_gather(g, par):
        return pltpu.make_async_copy(
            table_hbm.at[idx_v.at[pl.ds(g * _RG, _RG)]],
            gbuf.at[par], gsems[par])

    def g_scat(g, par):
        return pltpu.make_async_copy(
            gbuf.at[par], out_hbm.at[pl.ds(base + g * _RG, _RG)], ssems[par])

    def f_scat(i, par):
        return pltpu.make_async_copy(
            fbuf.at[pl.ds(par * _RF, _RF)],
            out_hbm.at[pl.ds(base + _GROWS + i * _RF, _RF)], fsems[par])

    def step(i, carry):
        par_f = i % 2

        @pl.when(i >= 2)
        def _():
            @pl.when(par_f == 0)
            def _():
                f_scat(i - 2, 0).wait()

            @pl.when(par_f == 1)
            def _():
                f_scat(i - 2, 1).wait()

        @pl.when(i < _NCHG)
        def _():
            g = i
            par_g = g % 2

            @pl.when(g == 0)
            def _():
                g_gather(0, 0).start()

            @pl.when(par_g == 0)
            def _():
                g_gather(g, 0).wait()
                g_scat(g, 0).start()

            @pl.when(par_g == 1)
            def _():
                g_gather(g, 1).wait()
                g_scat(g, 1).start()

            @pl.when(g >= 1)
            def _():
                @pl.when(par_g == 0)
                def _():
                    g_scat(g - 1, 1).wait()

                @pl.when(par_g == 1)
                def _():
                    g_scat(g - 1, 0).wait()

            @pl.when(g + 1 < _NCHG)
            def _():
                @pl.when(par_g == 0)
                def _():
                    g_gather(g + 1, 1).start()

                @pl.when(par_g == 1)
                def _():
                    g_gather(g + 1, 0).start()

        @plsc.parallel_loop(0, _RF)
        def _(r):
            rid = plsc.load_gather(
                idx_v, [jnp.full((16,), 0, jnp.int32) + (_GROWS + i * _RF + r)])
            row2d = par_f * _RF + r
            for j in range(D_MODEL // 16):
                vals = plsc.load_gather(table_v, [rid, iota + (j * 16)])
                fbuf[row2d, pl.ds(j * 16, 16)] = vals

        @pl.when(par_f == 0)
        def _():
            f_scat(i, 0).start()

        @pl.when(par_f == 1)
        def _():
            f_scat(i, 1).start()

        return carry

    lax.fori_loop(0, _NCHF, step, 0)
    f_scat(_NCHF - 2, 0).wait()
    f_scat(_NCHF - 1, 1).wait()
    g_scat(_NCHG - 1, (_NCHG - 1) % 2).wait()


@jax.jit
def _sc_lookup(ids_flat, table_rep):
    mesh = plsc.VectorSubcoreMesh(
        core_axis_name="c", subcore_axis_name="s",
        num_cores=_NC, num_subcores=_NS)
    f = functools.partial(
        pl.kernel,
        out_type=jax.ShapeDtypeStruct((_TOTAL, D_MODEL), jnp.float32),
        mesh=mesh,
        scratch_types=[
            pltpu.VMEM((_BPW,), jnp.int32),
            pltpu.VMEM((NUM_EMB, D_MODEL), jnp.float32),
            pltpu.VMEM((2, _RG, D_MODEL), jnp.float32),
            pltpu.VMEM((2 * _RF, D_MODEL), jnp.float32),
            pltpu.SemaphoreType.DMA,
            pltpu.SemaphoreType.DMA,
            pltpu.SemaphoreType.DMA,
            pltpu.SemaphoreType.DMA,
            pltpu.SemaphoreType.DMA,
            pltpu.SemaphoreType.DMA,
        ],
        compiler_params=pltpu.CompilerParams(needs_layout_passes=False),
    )(_sc_body)
    return f(ids_flat, table_rep)


def kernel(postion_ids, table):
    B, S = postion_ids.shape
    ids_flat = postion_ids.reshape(B * S).astype(jnp.int32)
    table_rep = jnp.tile(table, (_NW, 1))
    out = _sc_lookup(ids_flat, table_rep)
    return out.reshape(B, S, D_MODEL)


# speedup vs baseline: 1.1162x; 1.0879x over previous
import functools

import jax
import jax.numpy as jnp
from jax import lax
from jax.experimental import pallas as pl
from jax.experimental.pallas import tpu as pltpu
from jax.experimental.pallas import tpu_sc as plsc

D_MODEL = 1024
NUM_EMB = 4

_NC = 2
_NS = 16
_NW = _NC * _NS

_TOTAL = 4 * 8192
_BPW = _TOTAL // _NW

_GROWS = 512
_RG = 32
_NCHG = _GROWS // _RG

_FROWS = _BPW - _GROWS
_RF = 16
_NCHF = _FROWS // _RF


def _sc_body(ids_hbm, table_hbm, tflat_hbm, out_hbm, idx_v, table_v, gbuf,
             fbuf, gsemA, gsemB, ssemA, ssemB, fsemA, fsemB):
    sid = lax.axis_index("s")
    wid = sid * _NC + lax.axis_index("c")
    base = wid * _BPW

    pltpu.sync_copy(tflat_hbm, table_v)
    pltpu.sync_copy(ids_hbm.at[pl.ds(base, _BPW)], idx_v)
    off = wid * NUM_EMB
    for j in range(_GROWS // 16):
        sl = pl.ds(j * 16, 16)
        idx_v[sl] = idx_v[sl] + off

    iota = lax.iota(jnp.int32, 16)
    gsems = (gsemA, gsemB)
    ssems = (ssemA, ssemB)
    fsems = (fsemA, fsemB)

    def g_gather(g, par):
        return pltpu.make_async_copy(
            table_hbm.at[idx_v.at[pl.ds(g * _RG, _RG)]],
            gbuf.at[par], gsems[par])

    def g_scat(g, par):
        return pltpu.make_async_copy(
            gbuf.at[par], out_hbm.at[pl.ds(base + g * _RG, _RG)], ssems[par])

    def f_scat(i, par):
        return pltpu.make_async_copy(
            fbuf.at[pl.ds(par * _RF, _RF)],
            out_hbm.at[pl.ds(base + _GROWS + i * _RF, _RF)], fsems[par])

    def step(i, carry):
        par_f = i % 2

        @pl.when(i >= 2)
        def _():
            @pl.when(par_f == 0)
            def _():
                f_scat(i - 2, 0).wait()

            @pl.when(par_f == 1)
            def _():
                f_scat(i - 2, 1).wait()

        @pl.when(par_f == 0)
        def _():
            g = i // 2
            par_g = g % 2

            @pl.when(g == 0)
            def _():
                g_gather(0, 0).start()

            @pl.when(par_g == 0)
            def _():
                g_gather(g, 0).wait()
                g_scat(g, 0).start()

            @pl.when(par_g == 1)
            def _():
                g_gather(g, 1).wait()
                g_scat(g, 1).start()

            @pl.when(g >= 1)
            def _():
                @pl.when(par_g == 0)
                def _():
                    g_scat(g - 1, 1).wait()

                @pl.when(par_g == 1)
                def _():
                    g_scat(g - 1, 0).wait()

            @pl.when(g + 1 < _NCHG)
            def _():
                @pl.when(par_g == 0)
                def _():
                    g_gather(g + 1, 1).start()

                @pl.when(par_g == 1)
                def _():
                    g_gather(g + 1, 0).start()

        @plsc.parallel_loop(0, _RF)
        def _(r):
            rid = plsc.load_gather(
                idx_v, [jnp.full((16,), 0, jnp.int32) + (_GROWS + i * _RF + r)])
            src0 = rid * D_MODEL + iota
            row2d = par_f * _RF + r
            for j in range(D_MODEL // 16):
                vals = plsc.load_gather(table_v, [src0 + (j * 16)])
                fbuf[row2d, pl.ds(j * 16, 16)] = vals

        @pl.when(par_f == 0)
        def _():
            f_scat(i, 0).start()

        @pl.when(par_f == 1)
        def _():
            f_scat(i, 1).start()

        return carry

    lax.fori_loop(0, _NCHF, step, 0)
    f_scat(_NCHF - 2, 0).wait()
    f_scat(_NCHF - 1, 1).wait()
    g_scat(_NCHG - 1, (_NCHG - 1) % 2).wait()


@jax.jit
def _sc_lookup(ids_flat, table_rep, table_flat):
    mesh = plsc.VectorSubcoreMesh(
        core_axis_name="c", subcore_axis_name="s",
        num_cores=_NC, num_subcores=_NS)
    f = functools.partial(
        pl.kernel,
        out_type=jax.ShapeDtypeStruct((_TOTAL, D_MODEL), jnp.float32),
        mesh=mesh,
        scratch_types=[
            pltpu.VMEM((_BPW,), jnp.int32),
            pltpu.VMEM((NUM_EMB * D_MODEL,), jnp.float32),
            pltpu.VMEM((2, _RG, D_MODEL), jnp.float32),
            pltpu.VMEM((2 * _RF, D_MODEL), jnp.float32),
            pltpu.SemaphoreType.DMA,
            pltpu.SemaphoreType.DMA,
            pltpu.SemaphoreType.DMA,
            pltpu.SemaphoreType.DMA,
            pltpu.SemaphoreType.DMA,
            pltpu.SemaphoreType.DMA,
        ],
        compiler_params=pltpu.CompilerParams(needs_layout_passes=False),
    )(_sc_body)
    return f(ids_flat, table_rep, table_flat)


def kernel(postion_ids, table):
    B, S = postion_ids.shape
    ids_flat = postion_ids.reshape(B * S).astype(jnp.int32)
    table_rep = jnp.tile(table, (_NW, 1))
    out = _sc_lookup(ids_flat, table_rep, table.reshape(NUM_EMB * D_MODEL))
    return out.reshape(B, S, D_MODEL)


# speedup vs baseline: 1.1408x; 1.0220x over previous
import functools

import jax
import jax.numpy as jnp
from jax import lax
from jax.experimental import pallas as pl
from jax.experimental.pallas import tpu as pltpu
from jax.experimental.pallas import tpu_sc as plsc

D_MODEL = 1024
NUM_EMB = 4

_NC = 2
_NS = 16
_NW = _NC * _NS

_TOTAL = 4 * 8192
_BPW = _TOTAL // _NW

_GROWS = 512
_RG = 32
_NCHG = _GROWS // _RG

_FROWS = _BPW - _GROWS
_RF = 16
_NCHF = _FROWS // _RF


def _sc_body(ids_hbm, table_hbm, out_hbm, idx_v, table_v, gbuf, fbuf,
             gsemA, gsemB, ssemA, ssemB, fsemA, fsemB):
    sid = lax.axis_index("s")
    wid = sid * _NC + lax.axis_index("c")
    base = wid * _BPW

    pltpu.sync_copy(table_hbm.at[pl.ds(0, NUM_EMB)], table_v)
    pltpu.sync_copy(ids_hbm.at[pl.ds(base, _BPW)], idx_v)
    off = wid * NUM_EMB
    for j in range(_GROWS // 16):
        sl = pl.ds(j * 16, 16)
        idx_v[sl] = idx_v[sl] + off

    iota = lax.iota(jnp.int32, 16)
    gsems = (gsemA, gsemB)
    ssems = (ssemA, ssemB)
    fsems = (fsemA, fsemB)

    def g_gather(g, par):
        return pltpu.make_async_copy(
            table_hbm.at[idx_v.at[pl.ds(g * _RG, _RG)]],
            gbuf.at[par], gsems[par])

    def g_scat(g, par):
        return pltpu.make_async_copy(
            gbuf.at[par], out_hbm.at[pl.ds(base + g * _RG, _RG)], ssems[par])

    def f_scat(i, par):
        return pltpu.make_async_copy(
            fbuf.at[pl.ds(par * _RF, _RF)],
            out_hbm.at[pl.ds(base + _GROWS + i * _RF, _RF)], fsems[par])

    def step(i, carry):
        par_f = i % 2

        @pl.when(i >= 2)
        def _():
            @pl.when(par_f == 0)
            def _():
                f_scat(i - 2, 0).wait()

            @pl.when(par_f == 1)
            def _():
                f_scat(i - 2, 1).wait()

        @pl.when(par_f == 0)
        def _():
            g = i // 2
            par_g = g % 2

            @pl.when(g == 0)
            def _():
                g_gather(0, 0).start()

            @pl.when(par_g == 0)
            def _():
                g_gather(g, 0).wait()
                g_scat(g, 0).start()

            @pl.when(par_g == 1)
            def _():
                g_gather(g, 1).wait()
                g_scat(g, 1).start()

            @pl.when(g >= 1)
            def _():
                @pl.when(par_g == 0)
                def _():
                    g_scat(g - 1, 1).wait()

                @pl.when(par_g == 1)
                def _():
                    g_scat(g - 1, 0).wait()

            @pl.when(g + 1 < _NCHG)
            def _():
                @pl.when(par_g == 0)
                def _():
                    g_gather(g + 1, 1).start()

                @pl.when(par_g == 1)
                def _():
                    g_gather(g + 1, 0).start()

        @plsc.parallel_loop(0, _RF)
        def _(r):
            rid = plsc.load_gather(
                idx_v, [jnp.full((16,), 0, jnp.int32) + (_GROWS + i * _RF + r)])
            row2d = par_f * _RF + r
            for j in range(D_MODEL // 16):
                vals = plsc.load_gather(table_v, [rid, iota + (j * 16)])
                fbuf[row2d, pl.ds(j * 16, 16)] = vals

        @pl.when(par_f == 0)
        def _():
            f_scat(i, 0).start()

        @pl.when(par_f == 1)
        def _():
            f_scat(i, 1).start()

        return carry

    lax.fori_loop(0, _NCHF, step, 0)
    f_scat(_NCHF - 2, 0).wait()
    f_scat(_NCHF - 1, 1).wait()
    g_scat(_NCHG - 1, (_NCHG - 1) % 2).wait()


@jax.jit
def _sc_lookup(ids_flat, table_rep):
    mesh = plsc.VectorSubcoreMesh(
        core_axis_name="c", subcore_axis_name="s",
        num_cores=_NC, num_subcores=_NS)
    f = functools.partial(
        pl.kernel,
        out_type=jax.ShapeDtypeStruct((_TOTAL, D_MODEL), jnp.float32),
        mesh=mesh,
        scratch_types=[
            pltpu.VMEM((_BPW,), jnp.int32),
            pltpu.VMEM((NUM_EMB, D_MODEL), jnp.float32),
            pltpu.VMEM((2, _RG, D_MODEL), jnp.float32),
            pltpu.VMEM((2 * _RF, D_MODEL), jnp.float32),
            pltpu.SemaphoreType.DMA,
            pltpu.SemaphoreType.DMA,
            pltpu.SemaphoreType.DMA,
            pltpu.SemaphoreType.DMA,
            pltpu.SemaphoreType.DMA,
            pltpu.SemaphoreType.DMA,
        ],
        compiler_params=pltpu.CompilerParams(needs_layout_passes=False),
    )(_sc_body)
    return f(ids_flat, table_rep)


def kernel(postion_ids, table):
    B, S = postion_ids.shape
    ids_flat = postion_ids.reshape(B * S).astype(jnp.int32)
    table_rep = jnp.tile(table, (_NW, 1))
    out = _sc_lookup(ids_flat, table_rep)
    return out.reshape(B, S, D_MODEL)
